# baseline (speedup 1.0000x reference)
import functools

import jax
import jax.numpy as jnp
from jax import lax
from jax.experimental import pallas as pl
from jax.experimental.pallas import tpu as pltpu

N_NODES = 10000
N_EDGES = 320000
D_NODE = 128
D_EDGE = 16
NUM_CLASSES = 32

BN = 2000
BE = 2560


def _gelu_exact(x):
    return 0.5 * x * (1.0 + lax.erf(x * 0.7071067811865476))


def _k0_body(nf_ref, wst_ref, a_ref, b_ref):
    x = nf_ref[...]
    a_ref[...] = jnp.dot(x, wst_ref[:, :D_NODE],
                         preferred_element_type=jnp.float32)
    b_ref[...] = jnp.dot(x, wst_ref[:, D_NODE:],
                         preferred_element_type=jnp.float32)


def _precompute_ab(nf, W_msg):
    wst = jnp.concatenate([W_msg[:, :D_NODE].T, W_msg[:, D_NODE:2 * D_NODE].T],
                          axis=1)
    return pl.pallas_call(
        _k0_body,
        grid=(N_NODES // BN,),
        in_specs=[
            pl.BlockSpec((BN, D_NODE), lambda i: (i, 0)),
            pl.BlockSpec((D_NODE, 2 * D_NODE), lambda i: (0, 0)),
        ],
        out_specs=[
            pl.BlockSpec((BN, D_NODE), lambda i: (i, 0)),
            pl.BlockSpec((BN, D_NODE), lambda i: (i, 0)),
        ],
        out_shape=[
            jax.ShapeDtypeStruct((N_NODES, D_NODE), jnp.float32),
            jax.ShapeDtypeStruct((N_NODES, D_NODE), jnp.float32),
        ],
    )(nf, wst)


def _k1_body(ef_ref, we_ref, wce_ref, bmsg_ref, bc1_ref, emsg_ref, e2_ref):
    e = ef_ref[...]
    emsg_ref[...] = jnp.dot(e, we_ref[...],
                            preferred_element_type=jnp.float32) + bmsg_ref[...]
    e2_ref[...] = jnp.dot(e, wce_ref[...],
                          preferred_element_type=jnp.float32) + bc1_ref[...]


def _precompute_edge_terms(ef, W_msg, b_msg, Wc1, bc1):
    we_t = W_msg[:, 2 * D_NODE:].T
    wce_t = Wc1[:, 2 * D_NODE:].T
    return pl.pallas_call(
        _k1_body,
        grid=(N_EDGES // BE,),
        in_specs=[
            pl.BlockSpec((BE, D_EDGE), lambda i: (i, 0)),
            pl.BlockSpec((D_EDGE, D_NODE), lambda i: (0, 0)),
            pl.BlockSpec((D_EDGE, D_EDGE), lambda i: (0, 0)),
            pl.BlockSpec((1, D_NODE), lambda i: (0, 0)),
            pl.BlockSpec((1, D_EDGE), lambda i: (0, 0)),
        ],
        out_specs=[
            pl.BlockSpec((BE, D_NODE), lambda i: (i, 0)),
            pl.BlockSpec((BE, D_EDGE), lambda i: (i, 0)),
        ],
        out_shape=[
            jax.ShapeDtypeStruct((N_EDGES, D_NODE), jnp.float32),
            jax.ShapeDtypeStruct((N_EDGES, D_EDGE), jnp.float32),
        ],
    )(ef, we_t, wce_t, b_msg.reshape(1, -1), bc1.reshape(1, -1))


def _k3_body(aggp_ref, nf_ref, wih_ref, whh_ref, bih_ref, bhh_ref, wc_ref,
             a2_ref, b2_ref):
    agg = aggp_ref[0] + aggp_ref[1]
    h = nf_ref[...]
    gi = jnp.dot(agg, wih_ref[...], preferred_element_type=jnp.float32) \
        + bih_ref[...]
    gh = jnp.dot(h, whh_ref[...], preferred_element_type=jnp.float32) \
        + bhh_ref[...]
    i_r = gi[:, :D_NODE]
    i_z = gi[:, D_NODE:2 * D_NODE]
    i_n = gi[:, 2 * D_NODE:]
    h_r = gh[:, :D_NODE]
    h_z = gh[:, D_NODE:2 * D_NODE]
    h_n = gh[:, 2 * D_NODE:]
    r = jax.nn.sigmoid(i_r + h_r)
    z = jax.nn.sigmoid(i_z + h_z)
    n = jnp.tanh(i_n + r * h_n)
    nf_up = (1.0 - z) * n + z * h
    ab2 = jnp.dot(nf_up, wc_ref[...], preferred_element_type=jnp.float32)
    a2_ref[...] = ab2[:, :D_EDGE]
    b2_ref[...] = ab2[:, D_EDGE:]


def _gru_and_cls_precompute(agg_partials, nf, W_ih, W_hh, b_ih, b_hh, Wc1):
    wc_st = jnp.concatenate([Wc1[:, :D_NODE].T, Wc1[:, D_NODE:2 * D_NODE].T],
                            axis=1)
    return pl.pallas_call(
        _k3_body,
        grid=(N_NODES // BN,),
        in_specs=[
            pl.BlockSpec((2, BN, D_NODE), lambda i: (0, i, 0)),
            pl.BlockSpec((BN, D_NODE), lambda i: (i, 0)),
            pl.BlockSpec((D_NODE, 3 * D_NODE), lambda i: (0, 0)),
            pl.BlockSpec((D_NODE, 3 * D_NODE), lambda i: (0, 0)),
            pl.BlockSpec((1, 3 * D_NODE), lambda i: (0, 0)),
            pl.BlockSpec((1, 3 * D_NODE), lambda i: (0, 0)),
            pl.BlockSpec((D_NODE, 2 * D_EDGE), lambda i: (0, 0)),
        ],
        out_specs=[
            pl.BlockSpec((BN, D_EDGE), lambda i: (i, 0)),
            pl.BlockSpec((BN, D_EDGE), lambda i: (i, 0)),
        ],
        out_shape=[
            jax.ShapeDtypeStruct((N_NODES, D_EDGE), jnp.float32),
            jax.ShapeDtypeStruct((N_NODES, D_EDGE), jnp.float32),
        ],
    )(agg_partials, nf, W_ih.T, W_hh.T, b_ih.reshape(1, -1),
      b_hh.reshape(1, -1), wc_st)


def _k5_body(z_ref, w2_ref, bc2_ref, out_ref):
    h = _gelu_exact(z_ref[...])
    out_ref[...] = jnp.dot(h, w2_ref[...],
                           preferred_element_type=jnp.float32) + bc2_ref[...]


def _cls_head(zcls, Wc2, bc2):
    return pl.pallas_call(
        _k5_body,
        grid=(N_EDGES // BE,),
        in_specs=[
            pl.BlockSpec((BE, D_EDGE), lambda i: (i, 0)),
            pl.BlockSpec((D_EDGE, NUM_CLASSES), lambda i: (0, 0)),
            pl.BlockSpec((1, NUM_CLASSES), lambda i: (0, 0)),
        ],
        out_specs=pl.BlockSpec((BE, NUM_CLASSES), lambda i: (i, 0)),
        out_shape=jax.ShapeDtypeStruct((N_EDGES, NUM_CLASSES), jnp.float32),
    )(zcls, Wc2.T, bc2.reshape(1, -1))


def kernel(node_features, edge_index, edge_features, W_msg, b_msg, W_ih, W_hh,
           b_ih, b_hh, Wc1, bc1, Wc2, bc2):
    src = edge_index[0]
    tgt = edge_index[1]

    a, b = _precompute_ab(node_features, W_msg)
    emsg, e2 = _precompute_edge_terms(edge_features, W_msg, b_msg, Wc1, bc1)

    msg = jax.nn.gelu(a[src] + b[tgt] + emsg, approximate=False)
    agg = jnp.zeros((N_NODES, D_NODE), jnp.float32).at[tgt].add(msg)
    agg_partials = jnp.stack([agg, jnp.zeros_like(agg)])

    a2, b2 = _gru_and_cls_precompute(agg_partials, node_features, W_ih, W_hh,
                                     b_ih, b_hh, Wc1)

    zcls = a2[src] + b2[tgt] + e2
    return _cls_head(zcls, Wc2, bc2)


# speedup vs baseline: 2.2675x; 2.2675x over previous
import functools

import jax
import jax.numpy as jnp
from jax import lax
from jax.experimental import pallas as pl
from jax.experimental.pallas import tpu as pltpu
from jax.experimental.pallas import tpu_sc as plsc

N_NODES = 10000
N_EDGES = 320000
D_NODE = 128
D_EDGE = 16
NUM_CLASSES = 32

BN = 2000
BE = 2560


def _gelu_exact(x):
    return 0.5 * x * (1.0 + lax.erf(x * 0.7071067811865476))


def _k0_body(nf_ref, wst_ref, a_ref, b_ref):
    x = nf_ref[...]
    a_ref[...] = jnp.dot(x, wst_ref[:, :D_NODE],
                         preferred_element_type=jnp.float32)
    b_ref[...] = jnp.dot(x, wst_ref[:, D_NODE:],
                         preferred_element_type=jnp.float32)


def _precompute_ab(nf, W_msg):
    wst = jnp.concatenate([W_msg[:, :D_NODE].T, W_msg[:, D_NODE:2 * D_NODE].T],
                          axis=1)
    return pl.pallas_call(
        _k0_body,
        grid=(N_NODES // BN,),
        in_specs=[
            pl.BlockSpec((BN, D_NODE), lambda i: (i, 0)),
            pl.BlockSpec((D_NODE, 2 * D_NODE), lambda i: (0, 0)),
        ],
        out_specs=[
            pl.BlockSpec((BN, D_NODE), lambda i: (i, 0)),
            pl.BlockSpec((BN, D_NODE), lambda i: (i, 0)),
        ],
        out_shape=[
            jax.ShapeDtypeStruct((N_NODES, D_NODE), jnp.float32),
            jax.ShapeDtypeStruct((N_NODES, D_NODE), jnp.float32),
        ],
    )(nf, wst)


def _k1_body(ef_ref, we_ref, wce_ref, bmsg_ref, bc1_ref, emsg_ref, e2_ref):
    e = ef_ref[...]
    emsg_ref[...] = jnp.dot(e, we_ref[...],
                            preferred_element_type=jnp.float32) + bmsg_ref[...]
    e2_ref[...] = jnp.dot(e, wce_ref[...],
                          preferred_element_type=jnp.float32) + bc1_ref[...]


def _precompute_edge_terms(ef, W_msg, b_msg, Wc1, bc1):
    we_t = W_msg[:, 2 * D_NODE:].T
    wce_t = Wc1[:, 2 * D_NODE:].T
    return pl.pallas_call(
        _k1_body,
        grid=(N_EDGES // BE,),
        in_specs=[
            pl.BlockSpec((BE, D_EDGE), lambda i: (i, 0)),
            pl.BlockSpec((D_EDGE, D_NODE), lambda i: (0, 0)),
            pl.BlockSpec((D_EDGE, D_EDGE), lambda i: (0, 0)),
            pl.BlockSpec((1, D_NODE), lambda i: (0, 0)),
            pl.BlockSpec((1, D_EDGE), lambda i: (0, 0)),
        ],
        out_specs=[
            pl.BlockSpec((BE, D_NODE), lambda i: (i, 0)),
            pl.BlockSpec((BE, D_EDGE), lambda i: (i, 0)),
        ],
        out_shape=[
            jax.ShapeDtypeStruct((N_EDGES, D_NODE), jnp.float32),
            jax.ShapeDtypeStruct((N_EDGES, D_EDGE), jnp.float32),
        ],
    )(ef, we_t, wce_t, b_msg.reshape(1, -1), bc1.reshape(1, -1))


def _k3_body(aggp_ref, nf_ref, wih_ref, whh_ref, bih_ref, bhh_ref, wc_ref,
             t2_ref):
    agg = aggp_ref[0] + aggp_ref[1]
    h = nf_ref[...]
    gi = jnp.dot(agg, wih_ref[...], preferred_element_type=jnp.float32) \
        + bih_ref[...]
    gh = jnp.dot(h, whh_ref[...], preferred_element_type=jnp.float32) \
        + bhh_ref[...]
    i_r = gi[:, :D_NODE]
    i_z = gi[:, D_NODE:2 * D_NODE]
    i_n = gi[:, 2 * D_NODE:]
    h_r = gh[:, :D_NODE]
    h_z = gh[:, D_NODE:2 * D_NODE]
    h_n = gh[:, 2 * D_NODE:]
    r = jax.nn.sigmoid(i_r + h_r)
    z = jax.nn.sigmoid(i_z + h_z)
    n = jnp.tanh(i_n + r * h_n)
    nf_up = (1.0 - z) * n + z * h
    ab2 = jnp.dot(nf_up, wc_ref[...], preferred_element_type=jnp.float32)
    t2_ref[...] = jnp.concatenate(
        [ab2, jnp.zeros((ab2.shape[0], D_NODE - 2 * D_EDGE), jnp.float32)],
        axis=1)


def _gru_and_cls_precompute(agg_partials, nf, W_ih, W_hh, b_ih, b_hh, Wc1):
    wc_st = jnp.concatenate([Wc1[:, :D_NODE].T, Wc1[:, D_NODE:2 * D_NODE].T],
                            axis=1)
    return pl.pallas_call(
        _k3_body,
        grid=(N_NODES // BN,),
        in_specs=[
            pl.BlockSpec((2, BN, D_NODE), lambda i: (0, i, 0)),
            pl.BlockSpec((BN, D_NODE), lambda i: (i, 0)),
            pl.BlockSpec((D_NODE, 3 * D_NODE), lambda i: (0, 0)),
            pl.BlockSpec((D_NODE, 3 * D_NODE), lambda i: (0, 0)),
            pl.BlockSpec((1, 3 * D_NODE), lambda i: (0, 0)),
            pl.BlockSpec((1, 3 * D_NODE), lambda i: (0, 0)),
            pl.BlockSpec((D_NODE, 2 * D_EDGE), lambda i: (0, 0)),
        ],
        out_specs=pl.BlockSpec((BN, D_NODE), lambda i: (i, 0)),
        out_shape=jax.ShapeDtypeStruct((N_NODES, D_NODE), jnp.float32),
    )(agg_partials, nf, W_ih.T, W_hh.T, b_ih.reshape(1, -1),
      b_hh.reshape(1, -1), wc_st)


def _k5_body(z_ref, w2_ref, bc2_ref, out_ref):
    h = _gelu_exact(z_ref[...])
    out_ref[...] = jnp.dot(h, w2_ref[...],
                           preferred_element_type=jnp.float32) + bc2_ref[...]


def _cls_head(zcls, Wc2, bc2):
    return pl.pallas_call(
        _k5_body,
        grid=(N_EDGES // BE,),
        in_specs=[
            pl.BlockSpec((BE, D_EDGE), lambda i: (i, 0)),
            pl.BlockSpec((D_EDGE, NUM_CLASSES), lambda i: (0, 0)),
            pl.BlockSpec((1, NUM_CLASSES), lambda i: (0, 0)),
        ],
        out_specs=pl.BlockSpec((BE, NUM_CLASSES), lambda i: (i, 0)),
        out_shape=jax.ShapeDtypeStruct((N_EDGES, NUM_CLASSES), jnp.float32),
    )(zcls, Wc2.T, bc2.reshape(1, -1))


NUM_SC = 2
NUM_SUB = 16
NW = NUM_SC * NUM_SUB
EPW = N_EDGES // NW
NPS = 624
NREM = N_NODES - NUM_SUB * NPS
CH = 80
NCH = EPW // CH
CH2 = 200
NCH2 = EPW // CH2

_SC_MESH = plsc.VectorSubcoreMesh(core_axis_name="c", subcore_axis_name="s")


def _gelu_sc(x):
    z = jnp.abs(x) * 0.7071067811865476
    t = 1.0 / (1.0 + 0.3275911 * z)
    y = ((((1.061405429 * t - 1.453152027) * t + 1.421413741) * t
          - 0.284496736) * t + 0.254829592) * t
    erf = 1.0 - y * jnp.exp(-z * z)
    phi_pos = 0.5 + 0.5 * erf
    phi = jnp.where(x >= 0.0, phi_pos, 1.0 - phi_pos)
    return x * phi


def _k2_body(a_hbm, b_hbm, e_hbm, src_hbm, tgt_hbm, zeros_hbm, out_hbm,
             src_v, tgt_v, a_rows, b_rows, e_rows, agg_sh, sem):
    c = lax.axis_index("c")
    s = lax.axis_index("s")
    wid = s * NUM_SC + c
    pltpu.sync_copy(zeros_hbm.at[pl.ds(s * NPS, NPS)],
                    agg_sh.at[pl.ds(s * NPS, NPS)])

    @pl.when(s == NUM_SUB - 1)
    def _():
        pltpu.sync_copy(zeros_hbm.at[pl.ds(NUM_SUB * NPS, NREM)],
                        agg_sh.at[pl.ds(NUM_SUB * NPS, NREM)])

    plsc.subcore_barrier()

    def chunk(k, carry):
        base = wid * EPW + k * CH
        pltpu.sync_copy(src_hbm.at[pl.ds(base, CH)], src_v)
        pltpu.sync_copy(tgt_hbm.at[pl.ds(base, CH)], tgt_v)
        pltpu.async_copy(a_hbm.at[src_v], a_rows, sem).wait()
        pltpu.async_copy(b_hbm.at[tgt_v], b_rows, sem).wait()
        pltpu.sync_copy(e_hbm.at[pl.ds(base, CH)], e_rows)

        def row(r, rcarry):
            for j in range(D_NODE // 16):
                sl = pl.ds(j * 16, 16)
                x = a_rows[r, sl] + b_rows[r, sl] + e_rows[r, sl]
                a_rows[r, sl] = _gelu_sc(x)
            return rcarry

        lax.fori_loop(0, CH, row, 0)
        pltpu.sync_copy(a_rows, agg_sh.at[tgt_v], add=True)
        return carry

    lax.fori_loop(0, NCH, chunk, 0)
    plsc.subcore_barrier()
    pltpu.sync_copy(agg_sh.at[pl.ds(s * NPS, NPS)],
                    out_hbm.at[c, pl.ds(s * NPS, NPS)])

    @pl.when(s == NUM_SUB - 1)
    def _():
        pltpu.sync_copy(agg_sh.at[pl.ds(NUM_SUB * NPS, NREM)],
                        out_hbm.at[c, pl.ds(NUM_SUB * NPS, NREM)])


def _msg_aggregate(a, b, emsg, src, tgt):
    zeros = jnp.zeros((N_NODES, D_NODE), jnp.float32)
    run = pl.kernel(
        _k2_body,
        out_type=jax.ShapeDtypeStruct((NUM_SC, N_NODES, D_NODE), jnp.float32),
        mesh=_SC_MESH,
        scratch_types=[
            pltpu.VMEM((CH,), jnp.int32),
            pltpu.VMEM((CH,), jnp.int32),
            pltpu.VMEM((CH, D_NODE), jnp.float32),
            pltpu.VMEM((CH, D_NODE), jnp.float32),
            pltpu.VMEM((CH, D_NODE), jnp.float32),
            pltpu.VMEM_SHARED((N_NODES, D_NODE), jnp.float32),
            pltpu.SemaphoreType.DMA,
        ],
    )
    return run(a, b, emsg, src, tgt, zeros)


def _k4_body(t2_hbm, e2_hbm, src_hbm, tgt_hbm, z_hbm,
             src_v, tgt_v, a_rows, b_rows, e2_rows, sem):
    c = lax.axis_index("c")
    s = lax.axis_index("s")
    wid = s * NUM_SC + c

    def chunk(k, carry):
        base = wid * EPW + k * CH2
        pltpu.sync_copy(src_hbm.at[pl.ds(base, CH2)], src_v)
        pltpu.sync_copy(tgt_hbm.at[pl.ds(base, CH2)], tgt_v)
        pltpu.async_copy(t2_hbm.at[src_v], a_rows, sem).wait()
        pltpu.async_copy(t2_hbm.at[tgt_v], b_rows, sem).wait()
        pltpu.sync_copy(e2_hbm.at[pl.ds(base, CH2)], e2_rows)

        def row(r, rcarry):
            sl = pl.ds(0, D_EDGE)
            e2_rows[r, sl] = (a_rows[r, sl] + b_rows[r, pl.ds(D_EDGE, D_EDGE)]
                              + e2_rows[r, sl])
            return rcarry

        lax.fori_loop(0, CH2, row, 0)
        pltpu.sync_copy(e2_rows, z_hbm.at[pl.ds(base, CH2)])
        return carry

    lax.fori_loop(0, NCH2, chunk, 0)


def _cls_gather(t2, e2, src, tgt):
    run = pl.kernel(
        _k4_body,
        out_type=jax.ShapeDtypeStruct((N_EDGES, D_EDGE), jnp.float32),
        mesh=_SC_MESH,
        scratch_types=[
            pltpu.VMEM((CH2,), jnp.int32),
            pltpu.VMEM((CH2,), jnp.int32),
            pltpu.VMEM((CH2, D_NODE), jnp.float32),
            pltpu.VMEM((CH2, D_NODE), jnp.float32),
            pltpu.VMEM((CH2, D_EDGE), jnp.float32),
            pltpu.SemaphoreType.DMA,
        ],
    )
    return run(t2, e2, src, tgt)


def kernel(node_features, edge_index, edge_features, W_msg, b_msg, W_ih, W_hh,
           b_ih, b_hh, Wc1, bc1, Wc2, bc2):
    src = edge_index[0]
    tgt = edge_index[1]

    a, b = _precompute_ab(node_features, W_msg)
    emsg, e2 = _precompute_edge_terms(edge_features, W_msg, b_msg, Wc1, bc1)

    agg_partials = _msg_aggregate(a, b, emsg, src, tgt)

    t2 = _gru_and_cls_precompute(agg_partials, node_features, W_ih, W_hh,
                                 b_ih, b_hh, Wc1)

    zcls = _cls_gather(t2, e2, src, tgt)
    return _cls_head(zcls, Wc2, bc2)


# speedup vs baseline: 2.4723x; 1.0903x over previous
import functools

import jax
import jax.numpy as jnp
from jax import lax
from jax.experimental import pallas as pl
from jax.experimental.pallas import tpu as pltpu
from jax.experimental.pallas import tpu_sc as plsc

N_NODES = 10000
N_EDGES = 320000
D_NODE = 128
D_EDGE = 16
NUM_CLASSES = 32

BN = 2000
BE = 2560


def _gelu_exact(x):
    return 0.5 * x * (1.0 + lax.erf(x * 0.7071067811865476))


def _k0_body(nf_ref, wst_ref, a_ref, b_ref):
    x = nf_ref[...]
    a_ref[...] = jnp.dot(x, wst_ref[:, :D_NODE],
                         preferred_element_type=jnp.float32)
    b_ref[...] = jnp.dot(x, wst_ref[:, D_NODE:],
                         preferred_element_type=jnp.float32)


def _precompute_ab(nf, W_msg):
    wst = jnp.concatenate([W_msg[:, :D_NODE].T, W_msg[:, D_NODE:2 * D_NODE].T],
                          axis=1)
    return pl.pallas_call(
        _k0_body,
        grid=(N_NODES // BN,),
        in_specs=[
            pl.BlockSpec((BN, D_NODE), lambda i: (i, 0)),
            pl.BlockSpec((D_NODE, 2 * D_NODE), lambda i: (0, 0)),
        ],
        out_specs=[
            pl.BlockSpec((BN, D_NODE), lambda i: (i, 0)),
            pl.BlockSpec((BN, D_NODE), lambda i: (i, 0)),
        ],
        out_shape=[
            jax.ShapeDtypeStruct((N_NODES, D_NODE), jnp.float32),
            jax.ShapeDtypeStruct((N_NODES, D_NODE), jnp.float32),
        ],
    )(nf, wst)


def _k1_body(ef_ref, we_ref, wce_ref, bmsg_ref, bc1_ref, emsg_ref, e2_ref):
    e = ef_ref[...]
    emsg_ref[...] = jnp.dot(e, we_ref[...],
                            preferred_element_type=jnp.float32) + bmsg_ref[...]
    e2_ref[...] = jnp.dot(e, wce_ref[...],
                          preferred_element_type=jnp.float32) + bc1_ref[...]


def _precompute_edge_terms(ef, W_msg, b_msg, Wc1, bc1):
    we_t = W_msg[:, 2 * D_NODE:].T
    wce_t = Wc1[:, 2 * D_NODE:].T
    return pl.pallas_call(
        _k1_body,
        grid=(N_EDGES // BE,),
        in_specs=[
            pl.BlockSpec((BE, D_EDGE), lambda i: (i, 0)),
            pl.BlockSpec((D_EDGE, D_NODE), lambda i: (0, 0)),
            pl.BlockSpec((D_EDGE, D_EDGE), lambda i: (0, 0)),
            pl.BlockSpec((1, D_NODE), lambda i: (0, 0)),
            pl.BlockSpec((1, D_EDGE), lambda i: (0, 0)),
        ],
        out_specs=[
            pl.BlockSpec((BE, D_NODE), lambda i: (i, 0)),
            pl.BlockSpec((BE, D_EDGE), lambda i: (i, 0)),
        ],
        out_shape=[
            jax.ShapeDtypeStruct((N_EDGES, D_NODE), jnp.float32),
            jax.ShapeDtypeStruct((N_EDGES, D_EDGE), jnp.float32),
        ],
    )(ef, we_t, wce_t, b_msg.reshape(1, -1), bc1.reshape(1, -1))


def _k3_body(aggp_ref, nf_ref, wih_ref, whh_ref, bih_ref, bhh_ref, wc_ref,
             t2_ref):
    agg = aggp_ref[0] + aggp_ref[1]
    h = nf_ref[...]
    gi = jnp.dot(agg, wih_ref[...], preferred_element_type=jnp.float32) \
        + bih_ref[...]
    gh = jnp.dot(h, whh_ref[...], preferred_element_type=jnp.float32) \
        + bhh_ref[...]
    i_r = gi[:, :D_NODE]
    i_z = gi[:, D_NODE:2 * D_NODE]
    i_n = gi[:, 2 * D_NODE:]
    h_r = gh[:, :D_NODE]
    h_z = gh[:, D_NODE:2 * D_NODE]
    h_n = gh[:, 2 * D_NODE:]
    r = jax.nn.sigmoid(i_r + h_r)
    z = jax.nn.sigmoid(i_z + h_z)
    n = jnp.tanh(i_n + r * h_n)
    nf_up = (1.0 - z) * n + z * h
    ab2 = jnp.dot(nf_up, wc_ref[...], preferred_element_type=jnp.float32)
    t2_ref[...] = jnp.concatenate(
        [ab2, jnp.zeros((ab2.shape[0], D_NODE - 2 * D_EDGE), jnp.float32)],
        axis=1)


def _gru_and_cls_precompute(agg_partials, nf, W_ih, W_hh, b_ih, b_hh, Wc1):
    wc_st = jnp.concatenate([Wc1[:, :D_NODE].T, Wc1[:, D_NODE:2 * D_NODE].T],
                            axis=1)
    return pl.pallas_call(
        _k3_body,
        grid=(N_NODES // BN,),
        in_specs=[
            pl.BlockSpec((2, BN, D_NODE), lambda i: (0, i, 0)),
            pl.BlockSpec((BN, D_NODE), lambda i: (i, 0)),
            pl.BlockSpec((D_NODE, 3 * D_NODE), lambda i: (0, 0)),
            pl.BlockSpec((D_NODE, 3 * D_NODE), lambda i: (0, 0)),
            pl.BlockSpec((1, 3 * D_NODE), lambda i: (0, 0)),
            pl.BlockSpec((1, 3 * D_NODE), lambda i: (0, 0)),
            pl.BlockSpec((D_NODE, 2 * D_EDGE), lambda i: (0, 0)),
        ],
        out_specs=pl.BlockSpec((BN, D_NODE), lambda i: (i, 0)),
        out_shape=jax.ShapeDtypeStruct((N_NODES, D_NODE), jnp.float32),
    )(agg_partials, nf, W_ih.T, W_hh.T, b_ih.reshape(1, -1),
      b_hh.reshape(1, -1), wc_st)


def _k5_body(z_ref, w2_ref, bc2_ref, out_ref):
    h = _gelu_exact(z_ref[...])
    out_ref[...] = jnp.dot(h, w2_ref[...],
                           preferred_element_type=jnp.float32) + bc2_ref[...]


def _cls_head(zcls, Wc2, bc2):
    return pl.pallas_call(
        _k5_body,
        grid=(N_EDGES // BE,),
        in_specs=[
            pl.BlockSpec((BE, D_EDGE), lambda i: (i, 0)),
            pl.BlockSpec((D_EDGE, NUM_CLASSES), lambda i: (0, 0)),
            pl.BlockSpec((1, NUM_CLASSES), lambda i: (0, 0)),
        ],
        out_specs=pl.BlockSpec((BE, NUM_CLASSES), lambda i: (i, 0)),
        out_shape=jax.ShapeDtypeStruct((N_EDGES, NUM_CLASSES), jnp.float32),
    )(zcls, Wc2.T, bc2.reshape(1, -1))


NUM_SC = 2
NUM_SUB = 16
NW = NUM_SC * NUM_SUB
EPW = N_EDGES // NW
NPS = 624
NREM = N_NODES - NUM_SUB * NPS
CH = 80
NCH = EPW // CH
CH2 = 200
NCH2 = EPW // CH2

_SC_MESH = plsc.VectorSubcoreMesh(core_axis_name="c", subcore_axis_name="s")


def _gelu_sc(x):
    x2 = x * x
    t = x * (-1.5957691216057308 - 0.07135481627357725 * x2)
    return x / (1.0 + jnp.exp(t))


def _k2_body(a_hbm, b_hbm, e_hbm, src_hbm, tgt_hbm, zeros_hbm, out_hbm,
             src_v, tgt_v, a_rows, b_rows, e_rows, agg_sh, sem):
    c = lax.axis_index("c")
    s = lax.axis_index("s")
    wid = s * NUM_SC + c
    pltpu.sync_copy(zeros_hbm.at[pl.ds(s * NPS, NPS)],
                    agg_sh.at[pl.ds(s * NPS, NPS)])

    @pl.when(s == NUM_SUB - 1)
    def _():
        pltpu.sync_copy(zeros_hbm.at[pl.ds(NUM_SUB * NPS, NREM)],
                        agg_sh.at[pl.ds(NUM_SUB * NPS, NREM)])

    plsc.subcore_barrier()

    def chunk(k, carry):
        base = wid * EPW + k * CH
        pltpu.sync_copy(src_hbm.at[pl.ds(base, CH)], src_v)
        pltpu.sync_copy(tgt_hbm.at[pl.ds(base, CH)], tgt_v)
        pltpu.async_copy(a_hbm.at[src_v], a_rows, sem).wait()
        pltpu.async_copy(b_hbm.at[tgt_v], b_rows, sem).wait()
        pltpu.sync_copy(e_hbm.at[pl.ds(base, CH)], e_rows)

        def row(r, rcarry):
            for j in range(D_NODE // 16):
                sl = pl.ds(j * 16, 16)
                x = a_rows[r, sl] + b_rows[r, sl] + e_rows[r, sl]
                a_rows[r, sl] = _gelu_sc(x)
            return rcarry

        lax.fori_loop(0, CH, row, 0)
        pltpu.sync_copy(a_rows, agg_sh.at[tgt_v], add=True)
        return carry

    lax.fori_loop(0, NCH, chunk, 0)
    plsc.subcore_barrier()
    pltpu.sync_copy(agg_sh.at[pl.ds(s * NPS, NPS)],
                    out_hbm.at[c, pl.ds(s * NPS, NPS)])

    @pl.when(s == NUM_SUB - 1)
    def _():
        pltpu.sync_copy(agg_sh.at[pl.ds(NUM_SUB * NPS, NREM)],
                        out_hbm.at[c, pl.ds(NUM_SUB * NPS, NREM)])


def _msg_aggregate(a, b, emsg, src, tgt):
    zeros = jnp.zeros((N_NODES, D_NODE), jnp.float32)
    run = pl.kernel(
        _k2_body,
        out_type=jax.ShapeDtypeStruct((NUM_SC, N_NODES, D_NODE), jnp.float32),
        mesh=_SC_MESH,
        scratch_types=[
            pltpu.VMEM((CH,), jnp.int32),
            pltpu.VMEM((CH,), jnp.int32),
            pltpu.VMEM((CH, D_NODE), jnp.float32),
            pltpu.VMEM((CH, D_NODE), jnp.float32),
            pltpu.VMEM((CH, D_NODE), jnp.float32),
            pltpu.VMEM_SHARED((N_NODES, D_NODE), jnp.float32),
            pltpu.SemaphoreType.DMA,
        ],
    )
    return run(a, b, emsg, src, tgt, zeros)


def _k4_body(t2_hbm, e2_hbm, src_hbm, tgt_hbm, z_hbm,
             src_v, tgt_v, a_rows, b_rows, e2_rows, sem):
    c = lax.axis_index("c")
    s = lax.axis_index("s")
    wid = s * NUM_SC + c

    def chunk(k, carry):
        base = wid * EPW + k * CH2
        pltpu.sync_copy(src_hbm.at[pl.ds(base, CH2)], src_v)
        pltpu.sync_copy(tgt_hbm.at[pl.ds(base, CH2)], tgt_v)
        pltpu.async_copy(t2_hbm.at[src_v], a_rows, sem).wait()
        pltpu.async_copy(t2_hbm.at[tgt_v], b_rows, sem).wait()
        pltpu.sync_copy(e2_hbm.at[pl.ds(base, CH2)], e2_rows)

        def row(r, rcarry):
            sl = pl.ds(0, D_EDGE)
            e2_rows[r, sl] = (a_rows[r, sl] + b_rows[r, pl.ds(D_EDGE, D_EDGE)]
                              + e2_rows[r, sl])
            return rcarry

        lax.fori_loop(0, CH2, row, 0)
        pltpu.sync_copy(e2_rows, z_hbm.at[pl.ds(base, CH2)])
        return carry

    lax.fori_loop(0, NCH2, chunk, 0)


def _cls_gather(t2, e2, src, tgt):
    run = pl.kernel(
        _k4_body,
        out_type=jax.ShapeDtypeStruct((N_EDGES, D_EDGE), jnp.float32),
        mesh=_SC_MESH,
        scratch_types=[
            pltpu.VMEM((CH2,), jnp.int32),
            pltpu.VMEM((CH2,), jnp.int32),
            pltpu.VMEM((CH2, D_NODE), jnp.float32),
            pltpu.VMEM((CH2, D_NODE), jnp.float32),
            pltpu.VMEM((CH2, D_EDGE), jnp.float32),
            pltpu.SemaphoreType.DMA,
        ],
    )
    return run(t2, e2, src, tgt)


def kernel(node_features, edge_index, edge_features, W_msg, b_msg, W_ih, W_hh,
           b_ih, b_hh, Wc1, bc1, Wc2, bc2):
    src = edge_index[0]
    tgt = edge_index[1]

    a, b = _precompute_ab(node_features, W_msg)
    emsg, e2 = _precompute_edge_terms(edge_features, W_msg, b_msg, Wc1, bc1)

    agg_partials = _msg_aggregate(a, b, emsg, src, tgt)

    t2 = _gru_and_cls_precompute(agg_partials, node_features, W_ih, W_hh,
                                 b_ih, b_hh, Wc1)

    zcls = _cls_gather(t2, e2, src, tgt)
    return _cls_head(zcls, Wc2, bc2)


# speedup vs baseline: 3.3920x; 1.3720x over previous
import functools

import jax
import jax.numpy as jnp
from jax import lax
from jax.experimental import pallas as pl
from jax.experimental.pallas import tpu as pltpu
from jax.experimental.pallas import tpu_sc as plsc

N_NODES = 10000
N_EDGES = 320000
D_NODE = 128
D_EDGE = 16
NUM_CLASSES = 32

BN = 2000
BE = 2560


def _gelu_exact(x):
    return 0.5 * x * (1.0 + lax.erf(x * 0.7071067811865476))


def _k0_body(nf_ref, wst_ref, a_ref, b_ref):
    x = nf_ref[...]
    a_ref[...] = jnp.dot(x, wst_ref[:, :D_NODE],
                         preferred_element_type=jnp.float32)
    b_ref[...] = jnp.dot(x, wst_ref[:, D_NODE:],
                         preferred_element_type=jnp.float32)


def _precompute_ab(nf, W_msg):
    wst = jnp.concatenate([W_msg[:, :D_NODE].T, W_msg[:, D_NODE:2 * D_NODE].T],
                          axis=1)
    return pl.pallas_call(
        _k0_body,
        grid=(N_NODES // BN,),
        in_specs=[
            pl.BlockSpec((BN, D_NODE), lambda i: (i, 0)),
            pl.BlockSpec((D_NODE, 2 * D_NODE), lambda i: (0, 0)),
        ],
        out_specs=[
            pl.BlockSpec((BN, D_NODE), lambda i: (i, 0)),
            pl.BlockSpec((BN, D_NODE), lambda i: (i, 0)),
        ],
        out_shape=[
            jax.ShapeDtypeStruct((N_NODES, D_NODE), jnp.float32),
            jax.ShapeDtypeStruct((N_NODES, D_NODE), jnp.float32),
        ],
    )(nf, wst)


def _k1_body(ef_ref, we_ref, wce_ref, bmsg_ref, bc1_ref, emsg_ref, e2_ref):
    e = ef_ref[...]
    emsg_ref[...] = jnp.dot(e, we_ref[...],
                            preferred_element_type=jnp.float32) + bmsg_ref[...]
    e2_ref[...] = jnp.dot(e, wce_ref[...],
                          preferred_element_type=jnp.float32) + bc1_ref[...]


def _precompute_edge_terms(ef, W_msg, b_msg, Wc1, bc1):
    we_t = W_msg[:, 2 * D_NODE:].T
    wce_t = Wc1[:, 2 * D_NODE:].T
    return pl.pallas_call(
        _k1_body,
        grid=(N_EDGES // BE,),
        in_specs=[
            pl.BlockSpec((BE, D_EDGE), lambda i: (i, 0)),
            pl.BlockSpec((D_EDGE, D_NODE), lambda i: (0, 0)),
            pl.BlockSpec((D_EDGE, D_EDGE), lambda i: (0, 0)),
            pl.BlockSpec((1, D_NODE), lambda i: (0, 0)),
            pl.BlockSpec((1, D_EDGE), lambda i: (0, 0)),
        ],
        out_specs=[
            pl.BlockSpec((BE, D_NODE), lambda i: (i, 0)),
            pl.BlockSpec((BE, D_EDGE), lambda i: (i, 0)),
        ],
        out_shape=[
            jax.ShapeDtypeStruct((N_EDGES, D_NODE), jnp.float32),
            jax.ShapeDtypeStruct((N_EDGES, D_EDGE), jnp.float32),
        ],
    )(ef, we_t, wce_t, b_msg.reshape(1, -1), bc1.reshape(1, -1))


def _k3_body(aggp_ref, nf_ref, wih_ref, whh_ref, bih_ref, bhh_ref, wc_ref,
             t2_ref):
    agg = aggp_ref[0] + aggp_ref[1]
    h = nf_ref[...]
    gi = jnp.dot(agg, wih_ref[...], preferred_element_type=jnp.float32) \
        + bih_ref[...]
    gh = jnp.dot(h, whh_ref[...], preferred_element_type=jnp.float32) \
        + bhh_ref[...]
    i_r = gi[:, :D_NODE]
    i_z = gi[:, D_NODE:2 * D_NODE]
    i_n = gi[:, 2 * D_NODE:]
    h_r = gh[:, :D_NODE]
    h_z = gh[:, D_NODE:2 * D_NODE]
    h_n = gh[:, 2 * D_NODE:]
    r = jax.nn.sigmoid(i_r + h_r)
    z = jax.nn.sigmoid(i_z + h_z)
    n = jnp.tanh(i_n + r * h_n)
    nf_up = (1.0 - z) * n + z * h
    ab2 = jnp.dot(nf_up, wc_ref[...], preferred_element_type=jnp.float32)
    t2_ref[...] = jnp.concatenate(
        [ab2, jnp.zeros((ab2.shape[0], D_NODE - 2 * D_EDGE), jnp.float32)],
        axis=1)


def _gru_and_cls_precompute(agg_partials, nf, W_ih, W_hh, b_ih, b_hh, Wc1):
    wc_st = jnp.concatenate([Wc1[:, :D_NODE].T, Wc1[:, D_NODE:2 * D_NODE].T],
                            axis=1)
    return pl.pallas_call(
        _k3_body,
        grid=(N_NODES // BN,),
        in_specs=[
            pl.BlockSpec((2, BN, D_NODE), lambda i: (0, i, 0)),
            pl.BlockSpec((BN, D_NODE), lambda i: (i, 0)),
            pl.BlockSpec((D_NODE, 3 * D_NODE), lambda i: (0, 0)),
            pl.BlockSpec((D_NODE, 3 * D_NODE), lambda i: (0, 0)),
            pl.BlockSpec((1, 3 * D_NODE), lambda i: (0, 0)),
            pl.BlockSpec((1, 3 * D_NODE), lambda i: (0, 0)),
            pl.BlockSpec((D_NODE, 2 * D_EDGE), lambda i: (0, 0)),
        ],
        out_specs=pl.BlockSpec((BN, D_NODE), lambda i: (i, 0)),
        out_shape=jax.ShapeDtypeStruct((N_NODES, D_NODE), jnp.float32),
    )(agg_partials, nf, W_ih.T, W_hh.T, b_ih.reshape(1, -1),
      b_hh.reshape(1, -1), wc_st)


def _k5_body(z_ref, w2_ref, bc2_ref, out_ref):
    h = _gelu_exact(z_ref[...])
    out_ref[...] = jnp.dot(h, w2_ref[...],
                           preferred_element_type=jnp.float32) + bc2_ref[...]


def _cls_head(zcls, Wc2, bc2):
    return pl.pallas_call(
        _k5_body,
        grid=(N_EDGES // BE,),
        in_specs=[
            pl.BlockSpec((BE, D_EDGE), lambda i: (i, 0)),
            pl.BlockSpec((D_EDGE, NUM_CLASSES), lambda i: (0, 0)),
            pl.BlockSpec((1, NUM_CLASSES), lambda i: (0, 0)),
        ],
        out_specs=pl.BlockSpec((BE, NUM_CLASSES), lambda i: (i, 0)),
        out_shape=jax.ShapeDtypeStruct((N_EDGES, NUM_CLASSES), jnp.float32),
    )(zcls, Wc2.T, bc2.reshape(1, -1))


NUM_SC = 2
NUM_SUB = 16
NW = NUM_SC * NUM_SUB
EPW = N_EDGES // NW
NPS = 624
NREM = N_NODES - NUM_SUB * NPS
CH = 40
NCH = EPW // CH
CH2 = 40
NCH2 = EPW // CH2

_SC_MESH = plsc.VectorSubcoreMesh(core_axis_name="c", subcore_axis_name="s")


def _gelu_sc(x):
    x2 = x * x
    t = x * (-1.5957691216057308 - 0.07135481627357725 * x2)
    return x / (1.0 + jnp.exp(t))


def _k2_body(a_hbm, b_hbm, e_hbm, src_hbm, tgt_hbm, zeros_hbm, out_hbm,
             src0, tgt0, src1, tgt1, a0, b0, e0, a1, b1, e1, agg_sh,
             isem0, isem1, gsem0, gsem1):
    c = lax.axis_index("c")
    s = lax.axis_index("s")
    wid = s * NUM_SC + c
    tile_base = wid * EPW
    pltpu.sync_copy(zeros_hbm.at[pl.ds(s * NPS, NPS)],
                    agg_sh.at[pl.ds(s * NPS, NPS)])

    @pl.when(s == NUM_SUB - 1)
    def _():
        pltpu.sync_copy(zeros_hbm.at[pl.ds(NUM_SUB * NPS, NREM)],
                        agg_sh.at[pl.ds(NUM_SUB * NPS, NREM)])

    plsc.subcore_barrier()

    slots = ((src0, tgt0, a0, b0, e0, isem0, gsem0),
             (src1, tgt1, a1, b1, e1, isem1, gsem1))

    def issue_idx(k, slot):
        src_v, tgt_v = slot[0], slot[1]
        isem = slot[5]
        base = tile_base + k * CH
        pltpu.async_copy(src_hbm.at[pl.ds(base, CH)], src_v, isem)
        pltpu.async_copy(tgt_hbm.at[pl.ds(base, CH)], tgt_v, isem)

    def wait_idx(slot):
        src_v, tgt_v = slot[0], slot[1]
        isem = slot[5]
        pltpu.make_async_copy(src_hbm.at[pl.ds(0, CH)], src_v, isem).wait()
        pltpu.make_async_copy(tgt_hbm.at[pl.ds(0, CH)], tgt_v, isem).wait()

    def issue_gather(k, slot):
        src_v, tgt_v, a_rows, b_rows, e_rows, _, gsem = slot
        base = tile_base + k * CH
        pltpu.async_copy(a_hbm.at[src_v], a_rows, gsem)
        pltpu.async_copy(b_hbm.at[tgt_v], b_rows, gsem)
        pltpu.async_copy(e_hbm.at[pl.ds(base, CH)], e_rows, gsem)

    def wait_gather(slot):
        a_rows, b_rows, e_rows, gsem = slot[2], slot[3], slot[4], slot[6]
        pltpu.make_async_copy(a_hbm.at[pl.ds(0, CH)], a_rows, gsem).wait()
        pltpu.make_async_copy(b_hbm.at[pl.ds(0, CH)], b_rows, gsem).wait()
        pltpu.make_async_copy(e_hbm.at[pl.ds(0, CH)], e_rows, gsem).wait()

    def compute_scatter(slot):
        tgt_v, a_rows, b_rows, e_rows = slot[1], slot[2], slot[3], slot[4]

        def row4(i, rcarry):
            for u in range(4):
                r = i * 4 + u
                for j in range(D_NODE // 16):
                    sl = pl.ds(j * 16, 16)
                    x = a_rows[r, sl] + b_rows[r, sl] + e_rows[r, sl]
                    a_rows[r, sl] = _gelu_sc(x)
            return rcarry

        lax.fori_loop(0, CH // 4, row4, 0)
        pltpu.sync_copy(a_rows, agg_sh.at[tgt_v], add=True)

    pltpu.sync_copy(src_hbm.at[pl.ds(tile_base, CH)], src0)
    pltpu.sync_copy(tgt_hbm.at[pl.ds(tile_base, CH)], tgt0)
    issue_gather(0, slots[0])
    issue_idx(1, slots[1])

    def pair(p, carry):
        k0 = 2 * p
        wait_gather(slots[0])
        wait_idx(slots[1])
        issue_gather(k0 + 1, slots[1])
        compute_scatter(slots[0])

        @pl.when(k0 + 2 < NCH)
        def _():
            issue_idx(k0 + 2, slots[0])

        wait_gather(slots[1])

        @pl.when(k0 + 2 < NCH)
        def _():
            wait_idx(slots[0])
            issue_gather(k0 + 2, slots[0])

        compute_scatter(slots[1])

        @pl.when(k0 + 3 < NCH)
        def _():
            issue_idx(k0 + 3, slots[1])

        return carry

    lax.fori_loop(0, NCH // 2, pair, 0)
    plsc.subcore_barrier()
    pltpu.sync_copy(agg_sh.at[pl.ds(s * NPS, NPS)],
                    out_hbm.at[c, pl.ds(s * NPS, NPS)])

    @pl.when(s == NUM_SUB - 1)
    def _():
        pltpu.sync_copy(agg_sh.at[pl.ds(NUM_SUB * NPS, NREM)],
                        out_hbm.at[c, pl.ds(NUM_SUB * NPS, NREM)])


def _msg_aggregate(a, b, emsg, src, tgt):
    zeros = jnp.zeros((N_NODES, D_NODE), jnp.float32)
    run = pl.kernel(
        _k2_body,
        out_type=jax.ShapeDtypeStruct((NUM_SC, N_NODES, D_NODE), jnp.float32),
        mesh=_SC_MESH,
        scratch_types=[
            pltpu.VMEM((CH,), jnp.int32),
            pltpu.VMEM((CH,), jnp.int32),
            pltpu.VMEM((CH,), jnp.int32),
            pltpu.VMEM((CH,), jnp.int32),
            pltpu.VMEM((CH, D_NODE), jnp.float32),
            pltpu.VMEM((CH, D_NODE), jnp.float32),
            pltpu.VMEM((CH, D_NODE), jnp.float32),
            pltpu.VMEM((CH, D_NODE), jnp.float32),
            pltpu.VMEM((CH, D_NODE), jnp.float32),
            pltpu.VMEM((CH, D_NODE), jnp.float32),
            pltpu.VMEM_SHARED((N_NODES, D_NODE), jnp.float32),
            pltpu.SemaphoreType.DMA,
            pltpu.SemaphoreType.DMA,
            pltpu.SemaphoreType.DMA,
            pltpu.SemaphoreType.DMA,
        ],
    )
    return run(a, b, emsg, src, tgt, zeros)


def _k4_body(t2_hbm, e2_hbm, src_hbm, tgt_hbm, z_hbm,
             src0, tgt0, a0, b0, e20, src1, tgt1, a1, b1, e21, sem0, sem1):
    c = lax.axis_index("c")
    s = lax.axis_index("s")
    wid = s * NUM_SC + c
    tile_base = wid * EPW

    slots = ((src0, tgt0, a0, b0, e20, sem0),
             (src1, tgt1, a1, b1, e21, sem1))

    def issue(k, slot):
        src_v, tgt_v, a_rows, b_rows, e2_rows, sem = slot
        base = tile_base + k * CH2
        pltpu.sync_copy(src_hbm.at[pl.ds(base, CH2)], src_v)
        pltpu.sync_copy(tgt_hbm.at[pl.ds(base, CH2)], tgt_v)
        pltpu.async_copy(t2_hbm.at[src_v], a_rows, sem)
        pltpu.async_copy(t2_hbm.at[tgt_v], b_rows, sem)
        pltpu.async_copy(e2_hbm.at[pl.ds(base, CH2)], e2_rows, sem)

    def finish(k, slot):
        a_rows, b_rows, e2_rows, sem = slot[2], slot[3], slot[4], slot[5]
        pltpu.make_async_copy(t2_hbm.at[pl.ds(0, CH2)], a_rows, sem).wait()
        pltpu.make_async_copy(t2_hbm.at[pl.ds(0, CH2)], b_rows, sem).wait()
        pltpu.make_async_copy(e2_hbm.at[pl.ds(0, CH2)], e2_rows, sem).wait()

        def row4(i, rcarry):
            for u in range(4):
                r = i * 4 + u
                sl = pl.ds(0, D_EDGE)
                e2_rows[r, sl] = (a_rows[r, sl]
                                  + b_rows[r, pl.ds(D_EDGE, D_EDGE)]
                                  + e2_rows[r, sl])
            return rcarry

        lax.fori_loop(0, CH2 // 4, row4, 0)
        pltpu.sync_copy(e2_rows, z_hbm.at[pl.ds(tile_base + k * CH2, CH2)])

    issue(0, slots[0])

    def pair(p, carry):
        k0 = 2 * p
        issue(k0 + 1, slots[1])
        finish(k0, slots[0])

        @pl.when(k0 + 2 < NCH2)
        def _():
            issue(k0 + 2, slots[0])

        finish(k0 + 1, slots[1])
        return carry

    lax.fori_loop(0, NCH2 // 2, pair, 0)


def _cls_gather(t2, e2, src, tgt):
    run = pl.kernel(
        _k4_body,
        out_type=jax.ShapeDtypeStruct((N_EDGES, D_EDGE), jnp.float32),
        mesh=_SC_MESH,
        scratch_types=[
            pltpu.VMEM((CH2,), jnp.int32),
            pltpu.VMEM((CH2,), jnp.int32),
            pltpu.VMEM((CH2, D_NODE), jnp.float32),
            pltpu.VMEM((CH2, D_NODE), jnp.float32),
            pltpu.VMEM((CH2, D_EDGE), jnp.float32),
            pltpu.VMEM((CH2,), jnp.int32),
            pltpu.VMEM((CH2,), jnp.int32),
            pltpu.VMEM((CH2, D_NODE), jnp.float32),
            pltpu.VMEM((CH2, D_NODE), jnp.float32),
            pltpu.VMEM((CH2, D_EDGE), jnp.float32),
            pltpu.SemaphoreType.DMA,
            pltpu.SemaphoreType.DMA,
        ],
    )
    return run(t2, e2, src, tgt)


def kernel(node_features, edge_index, edge_features, W_msg, b_msg, W_ih, W_hh,
           b_ih, b_hh, Wc1, bc1, Wc2, bc2):
    src = edge_index[0]
    tgt = edge_index[1]

    a, b = _precompute_ab(node_features, W_msg)
    emsg, e2 = _precompute_edge_terms(edge_features, W_msg, b_msg, Wc1, bc1)

    agg_partials = _msg_aggregate(a, b, emsg, src, tgt)

    t2 = _gru_and_cls_precompute(agg_partials, node_features, W_ih, W_hh,
                                 b_ih, b_hh, Wc1)

    zcls = _cls_gather(t2, e2, src, tgt)
    return _cls_head(zcls, Wc2, bc2)
